# Initial kernel scaffold; baseline (speedup 1.0000x reference)
import functools

import jax
import jax.numpy as jnp
from jax import lax
from jax.experimental import pallas as pl
from jax.experimental.pallas import tpu as pltpu
from jax.experimental.pallas import tpu_sc as plsc

N_PAD = 10240
EB = 128
NC, NS = 2, 16
NW = NC * NS
NPB = 79
EPW = NPB * EB
E_PAD = NW * EPW
RPS = N_PAD // NS
CHUNK = 128
N_GRAPHS = 64


def _sc_mesh():
    return plsc.VectorSubcoreMesh(core_axis_name="c", subcore_axis_name="s")


def _deg_body(dst_hbm, ones_hbm, zeros_hbm, out_hbm, idx_d, ones_v, buf, sem,
              acc_sh):
    c = lax.axis_index("c")
    s = lax.axis_index("s")
    wid = c * NS + s
    r0 = s * RPS

    pltpu.sync_copy(zeros_hbm, buf)

    def zloop(k, _):
        pltpu.sync_copy(buf, acc_sh.at[pl.ds(r0 + k * CHUNK, CHUNK)])
        return _

    lax.fori_loop(0, RPS // CHUNK, zloop, None)
    pltpu.sync_copy(ones_hbm, ones_v)
    plsc.subcore_barrier()

    e0 = wid * EPW

    def eloop(i, _):
        pltpu.sync_copy(dst_hbm.at[pl.ds(e0 + i * EB, EB)], idx_d)
        pltpu.sync_copy(ones_v, acc_sh.at[idx_d], add=True)
        return _

    lax.fori_loop(0, NPB, eloop, None)
    plsc.subcore_barrier()

    def cloop(k, _):
        r = r0 + k * CHUNK
        pltpu.sync_copy(acc_sh.at[pl.ds(r, CHUNK)], buf)
        pltpu.sync_copy(buf, out_hbm.at[c, pl.ds(r, CHUNK)])
        return _

    lax.fori_loop(0, RPS // CHUNK, cloop, None)


_deg_kernel = functools.partial(
    pl.kernel,
    out_type=jax.ShapeDtypeStruct((NC, N_PAD, 8), jnp.float32),
    mesh=_sc_mesh(),
    scratch_types=[
        pltpu.VMEM((EB,), jnp.int32),
        pltpu.VMEM((EB, 8), jnp.float32),
        pltpu.VMEM((CHUNK, 8), jnp.float32),
        pltpu.SemaphoreType.DMA,
        pltpu.VMEM_SHARED((N_PAD, 8), jnp.float32),
    ],
)(_deg_body)


def _agg_body(y_hbm, src_hbm, dst_hbm, zeros_hbm, out_hbm, idx_s, idx_d, rows,
              sem, acc_sh):
    c = lax.axis_index("c")
    s = lax.axis_index("s")
    wid = c * NS + s
    r0 = s * RPS

    pltpu.sync_copy(zeros_hbm, rows)

    def zloop(k, _):
        pltpu.sync_copy(rows, acc_sh.at[pl.ds(r0 + k * CHUNK, CHUNK)])
        return _

    lax.fori_loop(0, RPS // CHUNK, zloop, None)
    plsc.subcore_barrier()

    e0 = wid * EPW

    def eloop(i, _):
        base = e0 + i * EB
        pltpu.sync_copy(src_hbm.at[pl.ds(base, EB)], idx_s)
        pltpu.sync_copy(dst_hbm.at[pl.ds(base, EB)], idx_d)
        pltpu.async_copy(y_hbm.at[idx_s], rows, sem).wait()
        pltpu.sync_copy(rows, acc_sh.at[idx_d], add=True)
        return _

    lax.fori_loop(0, NPB, eloop, None)
    plsc.subcore_barrier()

    def cloop(k, _):
        r = r0 + k * CHUNK
        pltpu.sync_copy(acc_sh.at[pl.ds(r, CHUNK)], rows)
        pltpu.sync_copy(rows, out_hbm.at[c, pl.ds(r, CHUNK)])
        return _

    lax.fori_loop(0, RPS // CHUNK, cloop, None)


def _make_agg(d):
    return functools.partial(
        pl.kernel,
        out_type=jax.ShapeDtypeStruct((NC, N_PAD, d), jnp.float32),
        mesh=_sc_mesh(),
        scratch_types=[
            pltpu.VMEM((EB,), jnp.int32),
            pltpu.VMEM((EB,), jnp.int32),
            pltpu.VMEM((EB, d), jnp.float32),
            pltpu.SemaphoreType.DMA,
            pltpu.VMEM_SHARED((N_PAD, d), jnp.float32),
        ],
    )(_agg_body)


_agg64 = _make_agg(64)
_agg128 = _make_agg(128)


def _t1_body(x_ref, w_ref, deg_ref, y_ref, dinv_ref):
    deg = deg_ref[0, :, 0:1] + deg_ref[1, :, 0:1] + 1.0
    dinv = lax.rsqrt(deg)
    y_ref[...] = dinv * jnp.dot(x_ref[...], w_ref[...],
                                preferred_element_type=jnp.float32)
    dinv_ref[...] = jnp.broadcast_to(dinv, (N_PAD, 8))


def _t2_body(agg_ref, y_ref, dinv_ref, b_ref, w_ref, out_ref):
    dinv = dinv_ref[:, 0:1]
    h = jnp.maximum(dinv * (agg_ref[0] + agg_ref[1] + y_ref[...]) + b_ref[...],
                    0.0)
    out_ref[...] = dinv * jnp.dot(h, w_ref[...],
                                  preferred_element_type=jnp.float32)


def _t4_body(agg_ref, y_ref, dinv_ref, b_ref, batch_ref, wlin_ref, blin_ref,
             out_ref):
    dinv = dinv_ref[:, 0:1]
    h = jnp.maximum(dinv * (agg_ref[0] + agg_ref[1] + y_ref[...]) + b_ref[...],
                    0.0)
    gids = lax.broadcasted_iota(jnp.int32, (N_GRAPHS, N_PAD), 0)
    onehot = (batch_ref[...] == gids).astype(jnp.float32)
    cnts = jnp.sum(onehot, axis=1, keepdims=True)
    sums = jnp.dot(onehot, h, preferred_element_type=jnp.float32)
    pooled = sums / jnp.maximum(cnts, 1.0)
    out_ref[...] = jnp.dot(pooled, wlin_ref[...],
                           preferred_element_type=jnp.float32) + blin_ref[...]


def kernel(x, edge_index, batch, W1, b1, W2, b2, W3, b3, Wlin, blin):
    n = x.shape[0]
    e = edge_index.shape[1]

    fill = jnp.full((E_PAD - e,), N_PAD - 8, jnp.int32)
    src = jnp.concatenate([edge_index[0], fill])
    dst = jnp.concatenate([edge_index[1], fill])
    x_p = jnp.pad(x, ((0, N_PAD - n), (0, 0)))
    batch_p = jnp.pad(batch, (0, N_PAD - n),
                      constant_values=N_GRAPHS).reshape(1, N_PAD)

    ones8 = jnp.ones((EB, 8), jnp.float32)
    zeros8 = jnp.zeros((CHUNK, 8), jnp.float32)
    zeros64 = jnp.zeros((CHUNK, 64), jnp.float32)
    zeros128 = jnp.zeros((CHUNK, 128), jnp.float32)

    degraw = _deg_kernel(dst, ones8, zeros8)

    y1, dinv8 = pl.pallas_call(
        _t1_body,
        out_shape=[
            jax.ShapeDtypeStruct((N_PAD, 64), jnp.float32),
            jax.ShapeDtypeStruct((N_PAD, 8), jnp.float32),
        ],
    )(x_p, W1, degraw)

    agg1 = _agg64(y1, src, dst, zeros64)

    y2 = pl.pallas_call(
        _t2_body,
        out_shape=jax.ShapeDtypeStruct((N_PAD, 128), jnp.float32),
    )(agg1, y1, dinv8, b1.reshape(1, 64), W2)

    agg2 = _agg128(y2, src, dst, zeros128)

    y3 = pl.pallas_call(
        _t2_body,
        out_shape=jax.ShapeDtypeStruct((N_PAD, 64), jnp.float32),
    )(agg2, y2, dinv8, b2.reshape(1, 128), W3)

    agg3 = _agg64(y3, src, dst, zeros64)

    out = pl.pallas_call(
        _t4_body,
        out_shape=jax.ShapeDtypeStruct((N_GRAPHS, 1), jnp.float32),
    )(agg3, y3, dinv8, b3.reshape(1, 64), batch_p, Wlin, blin.reshape(1, 1))

    return out


# speedup vs baseline: 12.3018x; 12.3018x over previous
import functools

import jax
import jax.numpy as jnp
from jax import lax
from jax.experimental import pallas as pl
from jax.experimental.pallas import tpu as pltpu
from jax.experimental.pallas import tpu_sc as plsc

N_PAD = 10240
EB = 128
NC, NS = 2, 16
NW = NC * NS
NPB = 79
EPW = NPB * EB
E_PAD = NW * EPW
RPS = N_PAD // NS
CHUNK = 128
N_GRAPHS = 64


def _sc_mesh():
    return plsc.VectorSubcoreMesh(core_axis_name="c", subcore_axis_name="s")


def _deg_body(dst_hbm, ones_hbm, zeros_hbm, out_hbm, idx_d, ones_v, buf, sem,
              acc_sh):
    c = lax.axis_index("c")
    s = lax.axis_index("s")
    wid = c * NS + s
    r0 = s * RPS

    pltpu.sync_copy(zeros_hbm, buf)

    def zloop(k, _):
        pltpu.sync_copy(buf, acc_sh.at[pl.ds(r0 + k * CHUNK, CHUNK)])
        return _

    lax.fori_loop(0, RPS // CHUNK, zloop, None)
    pltpu.sync_copy(ones_hbm, ones_v)
    plsc.subcore_barrier()

    e0 = wid * EPW

    def eloop(i, _):
        pltpu.sync_copy(dst_hbm.at[pl.ds(e0 + i * EB, EB)], idx_d)
        pltpu.sync_copy(ones_v, acc_sh.at[idx_d], add=True)
        return _

    lax.fori_loop(0, NPB, eloop, None)
    plsc.subcore_barrier()

    def cloop(k, _):
        r = r0 + k * CHUNK
        pltpu.sync_copy(acc_sh.at[pl.ds(r, CHUNK)], buf)
        pltpu.sync_copy(buf, out_hbm.at[c, pl.ds(r, CHUNK)])
        return _

    lax.fori_loop(0, RPS // CHUNK, cloop, None)


_deg_kernel = functools.partial(
    pl.kernel,
    out_type=jax.ShapeDtypeStruct((NC, N_PAD, 8), jnp.float32),
    mesh=_sc_mesh(),
    compiler_params=pltpu.CompilerParams(use_tc_tiling_on_sc=False),
    scratch_types=[
        pltpu.VMEM((EB,), jnp.int32),
        pltpu.VMEM((EB, 8), jnp.float32),
        pltpu.VMEM((CHUNK, 8), jnp.float32),
        pltpu.SemaphoreType.DMA,
        pltpu.VMEM_SHARED((N_PAD, 8), jnp.float32),
    ],
)(_deg_body)


def _agg_body(y_hbm, src_hbm, dst_hbm, zeros_hbm, out_hbm, idx_s, idx_d, rows,
              sem, acc_sh):
    c = lax.axis_index("c")
    s = lax.axis_index("s")
    wid = c * NS + s
    r0 = s * RPS

    pltpu.sync_copy(zeros_hbm, rows)

    def zloop(k, _):
        pltpu.sync_copy(rows, acc_sh.at[pl.ds(r0 + k * CHUNK, CHUNK)])
        return _

    lax.fori_loop(0, RPS // CHUNK, zloop, None)
    plsc.subcore_barrier()

    e0 = wid * EPW

    def eloop(i, _):
        base = e0 + i * EB
        pltpu.sync_copy(src_hbm.at[pl.ds(base, EB)], idx_s)
        pltpu.sync_copy(dst_hbm.at[pl.ds(base, EB)], idx_d)
        pltpu.async_copy(y_hbm.at[idx_s], rows, sem).wait()
        pltpu.sync_copy(rows, acc_sh.at[idx_d], add=True)
        return _

    lax.fori_loop(0, NPB, eloop, None)
    plsc.subcore_barrier()

    def cloop(k, _):
        r = r0 + k * CHUNK
        pltpu.sync_copy(acc_sh.at[pl.ds(r, CHUNK)], rows)
        pltpu.sync_copy(rows, out_hbm.at[c, pl.ds(r, CHUNK)])
        return _

    lax.fori_loop(0, RPS // CHUNK, cloop, None)


def _make_agg(d):
    return functools.partial(
        pl.kernel,
        out_type=jax.ShapeDtypeStruct((NC, N_PAD, d), jnp.float32),
        mesh=_sc_mesh(),
        compiler_params=pltpu.CompilerParams(use_tc_tiling_on_sc=False),
        scratch_types=[
            pltpu.VMEM((EB,), jnp.int32),
            pltpu.VMEM((EB,), jnp.int32),
            pltpu.VMEM((EB, d), jnp.float32),
            pltpu.SemaphoreType.DMA,
            pltpu.VMEM_SHARED((N_PAD, d), jnp.float32),
        ],
    )(_agg_body)


_agg64 = _make_agg(64)
_agg128 = _make_agg(128)


def _t1_body(x_ref, w_ref, deg_ref, y_ref, dinv_ref):
    deg = deg_ref[0, :, 0:1] + deg_ref[1, :, 0:1] + 1.0
    dinv = lax.rsqrt(deg)
    y_ref[...] = dinv * jnp.dot(x_ref[...], w_ref[...],
                                preferred_element_type=jnp.float32)
    dinv_ref[...] = jnp.broadcast_to(dinv, (N_PAD, 8))


def _t2_body(agg_ref, y_ref, dinv_ref, b_ref, w_ref, out_ref):
    dinv = dinv_ref[:, 0:1]
    h = jnp.maximum(dinv * (agg_ref[0] + agg_ref[1] + y_ref[...]) + b_ref[...],
                    0.0)
    out_ref[...] = dinv * jnp.dot(h, w_ref[...],
                                  preferred_element_type=jnp.float32)


def _t4_body(agg_ref, y_ref, dinv_ref, b_ref, batch_ref, wlin_ref, blin_ref,
             out_ref):
    dinv = dinv_ref[:, 0:1]
    h = jnp.maximum(dinv * (agg_ref[0] + agg_ref[1] + y_ref[...]) + b_ref[...],
                    0.0)
    gids = lax.broadcasted_iota(jnp.int32, (N_GRAPHS, N_PAD), 0)
    onehot = (batch_ref[...] == gids).astype(jnp.float32)
    cnts = jnp.sum(onehot, axis=1, keepdims=True)
    sums = jnp.dot(onehot, h, preferred_element_type=jnp.float32)
    pooled = sums / jnp.maximum(cnts, 1.0)
    out_ref[...] = jnp.dot(pooled, wlin_ref[...],
                           preferred_element_type=jnp.float32) + blin_ref[...]


def kernel(x, edge_index, batch, W1, b1, W2, b2, W3, b3, Wlin, blin):
    n = x.shape[0]
    e = edge_index.shape[1]

    fill = jnp.full((E_PAD - e,), N_PAD - 8, jnp.int32)
    src = jnp.concatenate([edge_index[0], fill])
    dst = jnp.concatenate([edge_index[1], fill])
    x_p = jnp.pad(x, ((0, N_PAD - n), (0, 0)))
    batch_p = jnp.pad(batch, (0, N_PAD - n),
                      constant_values=N_GRAPHS).reshape(1, N_PAD)

    ones8 = jnp.ones((EB, 8), jnp.float32)
    zeros8 = jnp.zeros((CHUNK, 8), jnp.float32)
    zeros64 = jnp.zeros((CHUNK, 64), jnp.float32)
    zeros128 = jnp.zeros((CHUNK, 128), jnp.float32)

    degraw = _deg_kernel(dst, ones8, zeros8)

    y1, dinv8 = pl.pallas_call(
        _t1_body,
        out_shape=[
            jax.ShapeDtypeStruct((N_PAD, 64), jnp.float32),
            jax.ShapeDtypeStruct((N_PAD, 8), jnp.float32),
        ],
    )(x_p, W1, degraw)

    agg1 = _agg64(y1, src, dst, zeros64)

    y2 = pl.pallas_call(
        _t2_body,
        out_shape=jax.ShapeDtypeStruct((N_PAD, 128), jnp.float32),
    )(agg1, y1, dinv8, b1.reshape(1, 64), W2)

    agg2 = _agg128(y2, src, dst, zeros128)

    y3 = pl.pallas_call(
        _t2_body,
        out_shape=jax.ShapeDtypeStruct((N_PAD, 64), jnp.float32),
    )(agg2, y2, dinv8, b2.reshape(1, 128), W3)

    agg3 = _agg64(y3, src, dst, zeros64)

    out = pl.pallas_call(
        _t4_body,
        out_shape=jax.ShapeDtypeStruct((N_GRAPHS, 1), jnp.float32),
    )(agg3, y3, dinv8, b3.reshape(1, 64), batch_p, Wlin, blin.reshape(1, 1))

    return out
